# baseline (speedup 1.0000x reference)
import functools

import jax
import jax.numpy as jnp
from jax import lax
from jax.experimental import pallas as pl
from jax.experimental.pallas import tpu as pltpu
from jax.experimental.pallas import tpu_sc as plsc

_LANES = 16
_IDX_CHUNK = 128


@functools.lru_cache(maxsize=None)
def _make_sc_partials(batch, feat, num_classes):
    info = plsc.get_sparse_core_info()
    nw = info.num_cores * info.num_subcores
    assert batch % nw == 0
    b_per_w = batch // nw
    assert b_per_w % _IDX_CHUNK == 0 and feat % _LANES == 0
    n_chunks = b_per_w // _IDX_CHUNK
    mesh = plsc.VectorSubcoreMesh(core_axis_name="c", subcore_axis_name="s")

    @functools.partial(
        pl.kernel,
        out_type=jax.ShapeDtypeStruct((nw, _LANES), jnp.float32),
        mesh=mesh,
        scratch_types=[
            pltpu.VMEM((b_per_w,), jnp.int32),
            pltpu.VMEM((b_per_w, feat), jnp.float32),
            pltpu.VMEM((b_per_w, feat), jnp.float32),
            pltpu.VMEM((_LANES,), jnp.float32),
            pltpu.SemaphoreType.DMA,
        ],
        compiler_params=pltpu.CompilerParams(use_tc_tiling_on_sc=False),
    )
    def sc_partials(x_hbm, labels_hbm, centers_hbm, out_hbm,
                    idx_v, rows_v, x_v, acc_v, sem):
        wid = lax.axis_index("s") * info.num_cores + lax.axis_index("c")
        base = wid * b_per_w
        pltpu.sync_copy(labels_hbm.at[pl.ds(base, b_per_w)], idx_v)
        gathers = []
        for j in range(n_chunks):
            sl = pl.ds(j * _IDX_CHUNK, _IDX_CHUNK)
            gathers.append(
                pltpu.async_copy(centers_hbm.at[idx_v.at[sl]], rows_v.at[sl], sem))
        pltpu.sync_copy(x_hbm.at[pl.ds(base, b_per_w)], x_v)
        for g in gathers:
            g.wait()

        def body(r, acc):
            for c in range(feat // _LANES):
                sl = pl.ds(c * _LANES, _LANES)
                d = rows_v[r, sl] - x_v[r, sl]
                acc = acc + d * d
            return acc

        acc = lax.fori_loop(0, b_per_w, body, jnp.zeros((_LANES,), jnp.float32))
        acc_v[...] = acc
        pltpu.sync_copy(acc_v, out_hbm.at[wid])

    return sc_partials


def kernel(x, labels, centers, lr):
    batch, feat = x.shape
    partials = _make_sc_partials(batch, feat, centers.shape[0])(
        x, labels.astype(jnp.int32), centers)
    return jnp.sum(partials) / 2.0 / batch
